# Initial kernel scaffold; baseline (speedup 1.0000x reference)
import jax
import jax.numpy as jnp
from jax.experimental import pallas as pl


def kernel(spec):
    raise NotImplementedError("write your pallas kernel here")


# speedup vs baseline: 19.9008x; 19.9008x over previous
import jax
import jax.numpy as jnp
from jax.experimental import pallas as pl
from jax.experimental.pallas import tpu as pltpu

_G = 64
_BB = 256


def _twohot_block(spec_ref, out_ref):
    s = spec_ref[...]
    sc = jnp.clip(s, 0.0, _G - 1.0 - 1e-06)
    lower = jnp.floor(sc)
    frac = sc - lower
    il = lower.astype(jnp.int32)[:, None, :]
    f = frac[:, None, :]
    g = jax.lax.broadcasted_iota(jnp.int32, out_ref.shape, 1)
    out_ref[...] = jnp.where(g == il, 1.0 - f, jnp.where(g == il + 1, f, 0.0))


def kernel(spec):
    b, d = spec.shape
    return pl.pallas_call(
        _twohot_block,
        grid=(b // _BB,),
        in_specs=[pl.BlockSpec((_BB, d), lambda i: (i, 0))],
        out_specs=pl.BlockSpec((_BB, _G, d), lambda i: (i, 0, 0)),
        out_shape=jax.ShapeDtypeStruct((b, _G, d), jnp.float32),
        compiler_params=pltpu.CompilerParams(
            dimension_semantics=("parallel",),
        ),
    )(spec)
